# Initial kernel scaffold; baseline (speedup 1.0000x reference)
import jax
import jax.numpy as jnp
from jax.experimental import pallas as pl


def kernel(input_ids, token_type_ids, token_table, position_table, type_table, gamma, beta):
    raise NotImplementedError("write your pallas kernel here")


# speedup vs baseline: 1.3067x; 1.3067x over previous
import functools

import jax
import jax.numpy as jnp
from jax import lax
from jax.experimental import pallas as pl
from jax.experimental.pallas import tpu as pltpu
from jax.experimental.pallas import tpu_sc as plsc

B, L, V, P, T, H = 4096, 200, 100000, 256, 2, 64
EPS = 1e-12

NC, NS, LANES = 2, 16, 16
NW = NC * NS
N = B * L
ROWS_PER_W = N // NW
CHUNK = 1024
NCHUNK = ROWS_PER_W // CHUNK
SUB = 128
NSUB = CHUNK // SUB
GROUPS = CHUNK // LANES


def _body(ids_hbm, tt_hbm, tok_hbm, pos_hbm, typ_hbm, g_hbm, b_hbm, out_hbm,
          idx_v, tt_v, rows_v, pt_v, typ_v, g_v, b_v, scr_e, gsem):
    wid = lax.axis_index("s") * NC + lax.axis_index("c")
    iota16 = lax.iota(jnp.int32, 16)

    rows_2d = rows_v
    pltpu.sync_copy(pos_hbm.at[pl.ds(0, L)], rows_2d.at[pl.ds(0, L)])
    pltpu.sync_copy(typ_hbm, typ_v)
    pltpu.sync_copy(g_hbm, g_v)
    pltpu.sync_copy(b_hbm, b_v)

    t0 = [typ_v[0, pl.ds(16 * k, 16)] for k in range(4)]
    t1 = [typ_v[1, pl.ds(16 * k, 16)] for k in range(4)]

    def build_pt(l, _):
        for k in range(4):
            pv = rows_v[l, pl.ds(16 * k, 16)]
            pt_v[2 * l, pl.ds(16 * k, 16)] = pv + t0[k]
            pt_v[2 * l + 1, pl.ds(16 * k, 16)] = pv + t1[k]
        return 0

    lax.fori_loop(0, L, build_pt, 0, unroll=False)

    def do_chunk(c, _):
        base = pl.multiple_of(wid * ROWS_PER_W + c * CHUNK, CHUNK)
        brow = pl.multiple_of(base // SUB, NSUB)

        pltpu.sync_copy(ids_hbm.at[pl.ds(brow, NSUB)], idx_v)
        pltpu.sync_copy(tt_hbm.at[pl.ds(brow, NSUB)], tt_v)

        cps = [
            pltpu.async_copy(tok_hbm.at[idx_v.at[j]],
                             rows_2d.at[pl.ds(j * SUB, SUB)], gsem)
            for j in range(NSUB)
        ]
        for cp in cps:
            cp.wait()

        def do_group(g, _):
            rows16 = g * LANES + iota16
            t_vec = tt_v[g // 8, pl.ds((g % 8) * 16, 16)]
            l_vec = lax.rem(base + rows16, L)
            ptrow = l_vec * 2 + t_vec

            def pass1(h, carry):
                s, s2 = carry
                hs = jnp.full((16,), h, jnp.int32)
                tok = plsc.load_gather(rows_v, [rows16, hs])
                pt = plsc.load_gather(pt_v, [ptrow, hs])
                e = tok + pt
                scr_e[h, :] = e
                return s + e, s2 + e * e

            s, s2 = lax.fori_loop(
                0, H, pass1,
                (jnp.zeros((16,), jnp.float32), jnp.zeros((16,), jnp.float32)),
                unroll=4)

            mean = s * (1.0 / H)
            var = s2 * (1.0 / H) - mean * mean
            x = var + EPS
            i = plsc.bitcast(x, jnp.int32)
            i = jnp.int32(0x5F3759DF) - lax.shift_right_logical(i, 1)
            y = plsc.bitcast(i, jnp.float32)
            for _ in range(3):
                y = y * (1.5 - 0.5 * x * y * y)

            def pass2(h, _):
                hs = jnp.full((16,), h, jnp.int32)
                e = scr_e[h, :]
                gk = plsc.load_gather(g_v, [hs])
                bk = plsc.load_gather(b_v, [hs])
                o = (e - mean) * y * gk + bk
                plsc.store_scatter(rows_v, [rows16, hs], o)
                return 0

            lax.fori_loop(0, H, pass2, 0, unroll=4)
            return 0

        lax.fori_loop(0, GROUPS, do_group, 0, unroll=False)

        pltpu.sync_copy(rows_2d, out_hbm.at[pl.ds(base, CHUNK)])
        return 0

    lax.fori_loop(0, NCHUNK, do_chunk, 0, unroll=False)


@jax.jit
def _run(ids2d, tt2d, token_table, position_table, type_table, gamma, beta):
    mesh = plsc.VectorSubcoreMesh(core_axis_name="c", subcore_axis_name="s",
                                  num_cores=NC, num_subcores=NS)
    k = pl.kernel(
        _body,
        out_type=jax.ShapeDtypeStruct((N, H), jnp.float32),
        mesh=mesh,
        scratch_types=[
            pltpu.VMEM((NSUB, SUB), jnp.int32),
            pltpu.VMEM((NSUB, SUB), jnp.int32),
            pltpu.VMEM((CHUNK, H), jnp.float32),
            pltpu.VMEM((2 * L, H), jnp.float32),
            pltpu.VMEM((T, H), jnp.float32),
            pltpu.VMEM((H,), jnp.float32),
            pltpu.VMEM((H,), jnp.float32),
            pltpu.VMEM((H, LANES), jnp.float32),
            pltpu.SemaphoreType.DMA,
        ],
        compiler_params=pltpu.CompilerParams(needs_layout_passes=False,
                                             use_tc_tiling_on_sc=False),
    )
    return k(ids2d, tt2d, token_table, position_table, type_table, gamma, beta)


def kernel(input_ids, token_type_ids, token_table, position_table, type_table,
           gamma, beta):
    ids2d = input_ids.astype(jnp.int32).reshape(N // SUB, SUB)
    tt2d = token_type_ids.astype(jnp.int32).reshape(N // SUB, SUB)
    out = _run(ids2d, tt2d, token_table, position_table, type_table, gamma,
               beta)
    return out.reshape(B, L, H)


# speedup vs baseline: 1.8558x; 1.4202x over previous
import functools

import jax
import jax.numpy as jnp
from jax import lax
from jax.experimental import pallas as pl
from jax.experimental.pallas import tpu as pltpu
from jax.experimental.pallas import tpu_sc as plsc

B, L, V, P, T, H = 4096, 200, 100000, 256, 2, 64
EPS = 1e-12

NC, NS, LANES = 2, 16, 16
NW = NC * NS
N = B * L
ROWS_PER_W = N // NW
CHUNK = 1024
NCHUNK = ROWS_PER_W // CHUNK
SUB = 128
NSUB = CHUNK // SUB
GROUPS = CHUNK // LANES


def _body(ids_hbm, tt_hbm, tok_hbm, pos_hbm, typ_hbm, g_hbm, b_hbm, out_hbm,
          idx_v, tt_v, rows_v, pt_v, typ_v, g_v, b_v, scr_e, gsem):
    wid = lax.axis_index("s") * NC + lax.axis_index("c")
    iota16 = lax.iota(jnp.int32, 16)

    rows_2d = rows_v
    pltpu.sync_copy(pos_hbm.at[pl.ds(0, L)], rows_2d.at[pl.ds(0, L)])
    pltpu.sync_copy(typ_hbm, typ_v)
    pltpu.sync_copy(g_hbm, g_v)
    pltpu.sync_copy(b_hbm, b_v)

    t0 = [typ_v[0, pl.ds(16 * k, 16)] for k in range(4)]
    t1 = [typ_v[1, pl.ds(16 * k, 16)] for k in range(4)]

    @plsc.parallel_loop(0, L, unroll=4)
    def build_pt(l):
        for k in range(4):
            pv = rows_v[l, pl.ds(16 * k, 16)]
            pt_v[2 * l, pl.ds(16 * k, 16)] = pv + t0[k]
            pt_v[2 * l + 1, pl.ds(16 * k, 16)] = pv + t1[k]

    def do_chunk(c, _):
        base = pl.multiple_of(wid * ROWS_PER_W + c * CHUNK, CHUNK)
        brow = pl.multiple_of(base // SUB, NSUB)

        pltpu.sync_copy(ids_hbm.at[pl.ds(brow, NSUB)], idx_v)
        pltpu.sync_copy(tt_hbm.at[pl.ds(brow, NSUB)], tt_v)

        cps = [
            pltpu.async_copy(tok_hbm.at[idx_v.at[j]],
                             rows_2d.at[pl.ds(j * SUB, SUB)], gsem)
            for j in range(NSUB)
        ]
        for cp in cps:
            cp.wait()

        def do_group(g, _):
            rows16 = g * LANES + iota16
            t_vec = tt_v[g // 8, pl.ds((g % 8) * 16, 16)]
            l_vec = lax.rem(base + rows16, L)
            ptrow = l_vec * 2 + t_vec

            zero16 = jnp.zeros((16,), jnp.float32)

            @plsc.parallel_loop(0, H, unroll=8,
                                carry=(zero16, zero16))
            def pass1(h, carry):
                s, s2 = carry
                hs = jnp.full((16,), h, jnp.int32)
                tok = plsc.load_gather(rows_v, [rows16, hs])
                pt = plsc.load_gather(pt_v, [ptrow, hs])
                e = tok + pt
                scr_e[h, :] = e
                return s + e, s2 + e * e

            s, s2 = pass1

            mean = s * (1.0 / H)
            var = s2 * (1.0 / H) - mean * mean
            x = var + EPS
            i = plsc.bitcast(x, jnp.int32)
            i = jnp.int32(0x5F3759DF) - lax.shift_right_logical(i, 1)
            y = plsc.bitcast(i, jnp.float32)
            for _ in range(3):
                y = y * (1.5 - 0.5 * x * y * y)

            @plsc.parallel_loop(0, H, unroll=8)
            def pass2(h):
                hs = jnp.full((16,), h, jnp.int32)
                e = scr_e[h, :]
                gk = plsc.load_gather(g_v, [hs])
                bk = plsc.load_gather(b_v, [hs])
                o = (e - mean) * y * gk + bk
                plsc.store_scatter(rows_v, [rows16, hs], o)

            return 0

        lax.fori_loop(0, GROUPS, do_group, 0, unroll=False)

        pltpu.sync_copy(rows_2d, out_hbm.at[pl.ds(base, CHUNK)])
        return 0

    lax.fori_loop(0, NCHUNK, do_chunk, 0, unroll=False)


@jax.jit
def _run(ids2d, tt2d, token_table, position_table, type_table, gamma, beta):
    mesh = plsc.VectorSubcoreMesh(core_axis_name="c", subcore_axis_name="s",
                                  num_cores=NC, num_subcores=NS)
    k = pl.kernel(
        _body,
        out_type=jax.ShapeDtypeStruct((N, H), jnp.float32),
        mesh=mesh,
        scratch_types=[
            pltpu.VMEM((NSUB, SUB), jnp.int32),
            pltpu.VMEM((NSUB, SUB), jnp.int32),
            pltpu.VMEM((CHUNK, H), jnp.float32),
            pltpu.VMEM((2 * L, H), jnp.float32),
            pltpu.VMEM((T, H), jnp.float32),
            pltpu.VMEM((H,), jnp.float32),
            pltpu.VMEM((H,), jnp.float32),
            pltpu.VMEM((H, LANES), jnp.float32),
            pltpu.SemaphoreType.DMA,
        ],
        compiler_params=pltpu.CompilerParams(needs_layout_passes=False,
                                             use_tc_tiling_on_sc=False),
    )
    return k(ids2d, tt2d, token_table, position_table, type_table, gamma, beta)


def kernel(input_ids, token_type_ids, token_table, position_table, type_table,
           gamma, beta):
    ids2d = input_ids.astype(jnp.int32).reshape(N // SUB, SUB)
    tt2d = token_type_ids.astype(jnp.int32).reshape(N // SUB, SUB)
    out = _run(ids2d, tt2d, token_table, position_table, type_table, gamma,
               beta)
    return out.reshape(B, L, H)


# speedup vs baseline: 5.3074x; 2.8599x over previous
import functools

import jax
import jax.numpy as jnp
from jax import lax
from jax.experimental import pallas as pl
from jax.experimental.pallas import tpu as pltpu
from jax.experimental.pallas import tpu_sc as plsc

B, L, V, P, T, H = 4096, 200, 100000, 256, 2, 64
EPS = 1e-12

NC, NS, LANES = 2, 16, 16
NW = NC * NS
N = B * L
ROWS_PER_W = N // NW
CHUNK = 1024
NCHUNK = ROWS_PER_W // CHUNK
SUB = 128
NSUB = CHUNK // SUB
GROUPS = CHUNK // LANES


def _body(ids_hbm, tt_hbm, tok_hbm, pos_hbm, typ_hbm, g_hbm, b_hbm, out_hbm,
          idx_v, tt_v, rows_v, pt_v, typ_v, g_v, b_v, scr_e, gsem):
    wid = lax.axis_index("s") * NC + lax.axis_index("c")
    iota16 = lax.iota(jnp.int32, 16)

    rows_2d = rows_v
    pltpu.sync_copy(pos_hbm.at[pl.ds(0, L)], rows_2d.at[pl.ds(0, L)])
    pltpu.sync_copy(typ_hbm, typ_v)
    pltpu.sync_copy(g_hbm, g_v)
    pltpu.sync_copy(b_hbm, b_v)

    t0 = [typ_v[0, pl.ds(16 * k, 16)] for k in range(4)]
    t1 = [typ_v[1, pl.ds(16 * k, 16)] for k in range(4)]

    @plsc.parallel_loop(0, L, unroll=4)
    def build_pt(l):
        for k in range(4):
            pv = rows_v[l, pl.ds(16 * k, 16)]
            pt_v[2 * l, pl.ds(16 * k, 16)] = pv + t0[k]
            pt_v[2 * l + 1, pl.ds(16 * k, 16)] = pv + t1[k]

    def do_chunk(c, _):
        base = pl.multiple_of(wid * ROWS_PER_W + c * CHUNK, CHUNK)
        brow = pl.multiple_of(base // SUB, NSUB)

        pltpu.sync_copy(ids_hbm.at[pl.ds(brow, NSUB)], idx_v)
        pltpu.sync_copy(tt_hbm.at[pl.ds(brow, NSUB)], tt_v)

        cps = [
            pltpu.async_copy(tok_hbm.at[idx_v.at[j]],
                             rows_2d.at[pl.ds(j * SUB, SUB)], gsem)
            for j in range(NSUB)
        ]
        for cp in cps:
            cp.wait()

        def do_group(g, _):
            rows16 = g * LANES + iota16
            t_vec = tt_v[g // 8, pl.ds((g % 8) * 16, 16)]
            l_vec = lax.rem(base + rows16, L)
            ptrow = l_vec * 2 + t_vec

            zero16 = jnp.zeros((16,), jnp.float32)

            @plsc.parallel_loop(0, H, unroll=8,
                                carry=(zero16, zero16))
            def pass1(h, carry):
                s, s2 = carry
                hd = (h + iota16) & (H - 1)
                tok = plsc.load_gather(rows_v, [rows16, hd])
                pt = plsc.load_gather(pt_v, [ptrow, hd])
                e = tok + pt
                scr_e[h, :] = e
                return s + e, s2 + e * e

            s, s2 = pass1

            mean = s * (1.0 / H)
            var = s2 * (1.0 / H) - mean * mean
            x = var + EPS
            i = plsc.bitcast(x, jnp.int32)
            i = jnp.int32(0x5F3759DF) - lax.shift_right_logical(i, 1)
            y = plsc.bitcast(i, jnp.float32)
            for _ in range(3):
                y = y * (1.5 - 0.5 * x * y * y)

            @plsc.parallel_loop(0, H, unroll=8)
            def pass2(h):
                hd = (h + iota16) & (H - 1)
                e = scr_e[h, :]
                gk = plsc.load_gather(g_v, [hd])
                bk = plsc.load_gather(b_v, [hd])
                o = (e - mean) * y * gk + bk
                plsc.store_scatter(rows_v, [rows16, hd], o)

            return 0

        lax.fori_loop(0, GROUPS, do_group, 0, unroll=False)

        pltpu.sync_copy(rows_2d, out_hbm.at[pl.ds(base, CHUNK)])
        return 0

    lax.fori_loop(0, NCHUNK, do_chunk, 0, unroll=False)


@jax.jit
def _run(ids2d, tt2d, token_table, position_table, type_table, gamma, beta):
    mesh = plsc.VectorSubcoreMesh(core_axis_name="c", subcore_axis_name="s",
                                  num_cores=NC, num_subcores=NS)
    k = pl.kernel(
        _body,
        out_type=jax.ShapeDtypeStruct((N, H), jnp.float32),
        mesh=mesh,
        scratch_types=[
            pltpu.VMEM((NSUB, SUB), jnp.int32),
            pltpu.VMEM((NSUB, SUB), jnp.int32),
            pltpu.VMEM((CHUNK, H), jnp.float32),
            pltpu.VMEM((2 * L, H), jnp.float32),
            pltpu.VMEM((T, H), jnp.float32),
            pltpu.VMEM((H,), jnp.float32),
            pltpu.VMEM((H,), jnp.float32),
            pltpu.VMEM((H, LANES), jnp.float32),
            pltpu.SemaphoreType.DMA,
        ],
        compiler_params=pltpu.CompilerParams(needs_layout_passes=False,
                                             use_tc_tiling_on_sc=False),
    )
    return k(ids2d, tt2d, token_table, position_table, type_table, gamma, beta)


def kernel(input_ids, token_type_ids, token_table, position_table, type_table,
           gamma, beta):
    ids2d = input_ids.astype(jnp.int32).reshape(N // SUB, SUB)
    tt2d = token_type_ids.astype(jnp.int32).reshape(N // SUB, SUB)
    out = _run(ids2d, tt2d, token_table, position_table, type_table, gamma,
               beta)
    return out.reshape(B, L, H)
